# baseline (speedup 1.0000x reference)
import functools

import jax
import jax.numpy as jnp
from jax import lax
from jax.experimental import pallas as pl
from jax.experimental.pallas import tpu as pltpu
from jax.experimental.pallas import tpu_sc as plsc


def _sc_gather(table, ids):
    V, D = table.shape
    B = ids.shape[0]
    info = plsc.get_sparse_core_info()
    nc, ns = info.num_cores, info.num_subcores
    nw = nc * ns
    b_per_w = B // nw

    mesh = plsc.VectorSubcoreMesh(core_axis_name="c", subcore_axis_name="s")

    @functools.partial(
        pl.kernel,
        mesh=mesh,
        out_type=jax.ShapeDtypeStruct((B, D), jnp.float32),
        scratch_types=[
            pltpu.VMEM((b_per_w,), jnp.int32),
            pltpu.VMEM((b_per_w, D), jnp.float32),
            pltpu.SemaphoreType.DMA,
        ],
    )
    def k(table_hbm, idx_hbm, out_hbm, idx_v, rows_v, sem):
        wid = lax.axis_index("s") * nc + lax.axis_index("c")
        base = wid * b_per_w
        pltpu.sync_copy(idx_hbm.at[pl.ds(base, b_per_w)], idx_v)
        pltpu.async_copy(table_hbm.at[idx_v], rows_v, sem).wait()
        pltpu.sync_copy(rows_v, out_hbm.at[pl.ds(base, b_per_w)])

    return k(table, ids)


def _assemble(lc, gc, b):
    B, W, DL = lc.shape
    DE = b.shape[0]
    BLK = 128
    grid = (B // BLK,)

    def body(lc_ref, gc_ref, b_ref, out_ref):
        gcb = gc_ref[:, :DE] + b_ref[...]
        out_ref[...] = jnp.concatenate(
            [lc_ref[...], jnp.broadcast_to(gcb[:, None, :], (BLK, W, DE))],
            axis=2,
        )

    return pl.pallas_call(
        body,
        grid=grid,
        in_specs=[
            pl.BlockSpec((BLK, W, DL), lambda i: (i, 0, 0)),
            pl.BlockSpec((BLK, gc.shape[1]), lambda i: (i, 0)),
            pl.BlockSpec((1, DE), lambda i: (0, 0)),
        ],
        out_specs=pl.BlockSpec((BLK, W, DL + DE), lambda i: (i, 0, 0)),
        out_shape=jax.ShapeDtypeStruct((B, W, DL + DE), lc.dtype),
    )(lc, gc, b.reshape(1, DE))


def kernel(lc, ids, W, b):
    table = jnp.transpose(W)
    pad = (-table.shape[1]) % 128
    if pad:
        table = jnp.pad(table, ((0, 0), (0, pad)))
    gc = _sc_gather(table, ids.astype(jnp.int32))
    return _assemble(lc, gc, b)


# speedup vs baseline: 1.0053x; 1.0053x over previous
import functools

import jax
import jax.numpy as jnp
from jax import lax
from jax.experimental import pallas as pl
from jax.experimental.pallas import tpu as pltpu
from jax.experimental.pallas import tpu_sc as plsc


def _sc_gather(table, ids):
    V, D = table.shape
    B = ids.shape[0]
    info = plsc.get_sparse_core_info()
    nc, ns = info.num_cores, info.num_subcores
    nw = nc * ns
    b_per_w = B // nw

    mesh = plsc.VectorSubcoreMesh(core_axis_name="c", subcore_axis_name="s")

    @functools.partial(
        pl.kernel,
        mesh=mesh,
        out_type=jax.ShapeDtypeStruct((B, D), jnp.float32),
        scratch_types=[
            pltpu.VMEM((b_per_w,), jnp.int32),
            pltpu.VMEM((b_per_w, D), jnp.float32),
            pltpu.SemaphoreType.DMA,
        ],
    )
    def k(table_hbm, idx_hbm, out_hbm, idx_v, rows_v, sem):
        wid = lax.axis_index("s") * nc + lax.axis_index("c")
        base = wid * b_per_w
        pltpu.sync_copy(idx_hbm.at[pl.ds(base, b_per_w)], idx_v)
        pltpu.async_copy(table_hbm.at[idx_v], rows_v, sem).wait()
        pltpu.sync_copy(rows_v, out_hbm.at[pl.ds(base, b_per_w)])

    return k(table, ids)


def _assemble(lc, gc, b):
    B, W, DL = lc.shape
    DE = b.shape[0]
    BLK = 256
    grid = (B // BLK,)

    def body(lc_ref, gc_ref, b_ref, out_ref):
        gcb = gc_ref[:, :DE] + b_ref[...]
        out_ref[...] = jnp.concatenate(
            [lc_ref[...], jnp.broadcast_to(gcb[:, None, :], (BLK, W, DE))],
            axis=2,
        )

    return pl.pallas_call(
        body,
        grid=grid,
        in_specs=[
            pl.BlockSpec((BLK, W, DL), lambda i: (i, 0, 0)),
            pl.BlockSpec((BLK, gc.shape[1]), lambda i: (i, 0)),
            pl.BlockSpec((1, DE), lambda i: (0, 0)),
        ],
        out_specs=pl.BlockSpec((BLK, W, DL + DE), lambda i: (i, 0, 0)),
        out_shape=jax.ShapeDtypeStruct((B, W, DL + DE), lc.dtype),
    )(lc, gc, b.reshape(1, DE))


def kernel(lc, ids, W, b):
    table = jnp.transpose(W)
    pad = (-table.shape[1]) % 128
    if pad:
        table = jnp.pad(table, ((0, 0), (0, pad)))
    gc = _sc_gather(table, ids.astype(jnp.int32))
    return _assemble(lc, gc, b)
